# Initial kernel scaffold; baseline (speedup 1.0000x reference)
import jax
import jax.numpy as jnp
from jax.experimental import pallas as pl


def kernel(x, edge_index, supernode_mask, edge_mask, batch, W1_0, b1_0, g_0, be_0, W2_0, b2_0, W1_1, b1_1, g_1, be_1, W2_1, b2_1, W1_2, b1_2, g_2, be_2, W2_2, b2_2, fW1, fb1, fW2, fb2):
    raise NotImplementedError("write your pallas kernel here")


# speedup vs baseline: 11.5201x; 11.5201x over previous
import functools
import math

import jax
import jax.numpy as jnp
from jax import lax
from jax.experimental import pallas as pl
from jax.experimental.pallas import tpu as pltpu
from jax.experimental.pallas import tpu_sc as plsc

N = 10000
E = 320000
D = 128
G = 64
C = 40

NC = 2
NS = 16
NW = NC * NS
LANES = 16

ACC_N = 10240
RPT = ACC_N // NS

BN_S = 1.0 / math.sqrt(1.0 + 1e-5)

BLK = 2000
NBLK = N // BLK


@functools.lru_cache(maxsize=None)
def _make_sc_seg_sum(weighted):
    mesh = plsc.VectorSubcoreMesh(
        core_axis_name="c", subcore_axis_name="s", num_cores=NC,
        num_subcores=NS)

    S = 4
    G = 2

    dt = jnp.float32
    K = 80
    CHUNKS = E // K
    CH_PER_W = -(-CHUNKS // NW)
    RFULL = RPT // K

    scratch = (
        [pltpu.VMEM((K,), jnp.int32)] * S +
        [pltpu.VMEM((K,), jnp.int32)] * S +
        [pltpu.VMEM((K, D), dt)] * S +
        ([pltpu.VMEM((K,), jnp.float32)] * S if weighted else []) +
        [pltpu.VMEM_SHARED((ACC_N, D), dt)] +
        [pltpu.SemaphoreType.DMA] * S +
        [pltpu.SemaphoreType.DMA] * S
    )

    def body(*refs):
        n_in = 4 if weighted else 3
        hbm = refs[:n_in]
        if weighted:
            x_hbm, src_hbm, dst_hbm, wt_hbm = hbm
        else:
            x_hbm, src_hbm, dst_hbm = hbm
            wt_hbm = None
        out0_hbm, out1_hbm = refs[n_in:n_in + 2]
        rest = list(refs[n_in + 2:])
        sidxs = tuple(rest[:S]); del rest[:S]
        didxs = tuple(rest[:S]); del rest[:S]
        rowss = tuple(rest[:S]); del rest[:S]
        if weighted:
            wts = tuple(rest[:S]); del rest[:S]
        acc = rest.pop(0)
        isems = tuple(rest[:S]); del rest[:S]
        gsems = tuple(rest[:S]); del rest[:S]

        c = lax.axis_index("c")
        s = lax.axis_index("s")
        w = s * NC + c
        abase = s * RPT

        def idx_fetch(j, slot):
            eb = jnp.minimum(w + j * NW, CHUNKS - 1) * K
            pltpu.async_copy(src_hbm.at[pl.ds(eb, K)], sidxs[slot],
                             isems[slot])
            pltpu.async_copy(dst_hbm.at[pl.ds(eb, K)], didxs[slot],
                             isems[slot])
            if weighted:
                pltpu.async_copy(wt_hbm.at[pl.ds(eb, K)], wts[slot],
                                 isems[slot])

        def idx_wait(slot):
            pltpu.make_async_copy(src_hbm.at[pl.ds(0, K)], sidxs[slot],
                                  isems[slot]).wait()
            pltpu.make_async_copy(dst_hbm.at[pl.ds(0, K)], didxs[slot],
                                  isems[slot]).wait()
            if weighted:
                pltpu.make_async_copy(wt_hbm.at[pl.ds(0, K)], wts[slot],
                                      isems[slot]).wait()

        def gather_start(slot):
            pltpu.async_copy(x_hbm.at[sidxs[slot]], rowss[slot], gsems[slot])

        def gather_wait(slot):
            pltpu.make_async_copy(x_hbm.at[sidxs[slot]], rowss[slot],
                                  gsems[slot]).wait()

        for k in range(S):
            idx_fetch(k, k)

        def zrow(i, carry):
            for cb in range(D // LANES):
                rowss[0][i, pl.ds(cb * LANES, LANES)] = jnp.zeros(
                    (LANES,), dt)
            return carry
        lax.fori_loop(0, K, zrow, 0)
        for j in range(RFULL):
            pltpu.async_copy(rowss[0], acc.at[pl.ds(abase + j * K, K)],
                             gsems[0])
        for j in range(RFULL):
            pltpu.make_async_copy(rowss[0], acc.at[pl.ds(abase + j * K, K)],
                                  gsems[0]).wait()
        plsc.subcore_barrier()

        for k in range(G):
            idx_wait(k)
            gather_start(k)

        def scale_rows(slot):
            def scale(grp, cr):
                wvec = wts[slot][pl.ds(grp * LANES, LANES)]
                for rs in range(LANES):
                    bw = jnp.take_along_axis(
                        wvec, jnp.full((LANES,), rs, jnp.int32), axis=0)
                    row = grp * LANES + rs
                    for cb in range(D // LANES):
                        sl = pl.ds(cb * LANES, LANES)
                        rowss[slot][row, sl] = rowss[slot][row, sl] * bw
                return cr
            lax.fori_loop(0, K // LANES, scale, 0)

        def step(j, slot):
            idx_wait((slot + G) % S)
            gather_start((slot + G) % S)
            gather_wait(slot)
            if weighted:
                scale_rows(slot)

            @pl.when(w + j * NW < CHUNKS)
            def _():
                pltpu.sync_copy(rowss[slot], acc.at[didxs[slot]], add=True)
            idx_fetch(j + S, slot)

        NSTEP = -(-CH_PER_W // S) * S

        def estep(i4, carry):
            for k in range(S):
                step(i4 * S + k, k)
            return carry
        lax.fori_loop(0, NSTEP // S, estep, 0)

        for k in range(G):
            gather_wait((NSTEP + k) % S)
        for k in range(G, S):
            idx_wait((NSTEP + k) % S)

        plsc.subcore_barrier()

        def wb_to(out_hbm):
            for j in range(RFULL):
                sl = pl.ds(abase + j * K, K)
                pltpu.async_copy(acc.at[sl], out_hbm.at[sl], gsems[0])
            for j in range(RFULL):
                sl = pl.ds(abase + j * K, K)
                pltpu.make_async_copy(acc.at[sl], out_hbm.at[sl],
                                      gsems[0]).wait()

        @pl.when(c == 0)
        def _():
            wb_to(out0_hbm)

        @pl.when(c == 1)
        def _():
            wb_to(out1_hbm)

    part = jax.ShapeDtypeStruct((ACC_N, D), dt)
    return pl.kernel(
        body,
        out_type=(part, part),
        mesh=mesh,
        scratch_types=scratch,
    )


def _sc_seg_w(x, src, dst, wt):
    return _make_sc_seg_sum(True)(x, src, dst, wt)


def _sc_seg(x, src, dst):
    return _make_sc_seg_sum(False)(x, src, dst)


_ROW = pl.BlockSpec((BLK, D), lambda i: (i, 0))


_FULL_W = pl.BlockSpec((D, D), lambda i: (0, 0))
_FULL_B = pl.BlockSpec((1, D), lambda i: (0, 0))


def _select_body(x_ref, p0_ref, p1_ref, m_ref, o_ref):
    m = m_ref[...]
    o_ref[...] = jnp.where(m != 0, p0_ref[...] + p1_ref[...], x_ref[...])


def _tc_select(x, p0, p1, maskf):
    return pl.pallas_call(
        _select_body,
        grid=(NBLK,),
        in_specs=[_ROW, _ROW, _ROW,
                  pl.BlockSpec((BLK, 1), lambda i: (i, 0))],
        out_specs=_ROW,
        out_shape=jax.ShapeDtypeStruct((N, D), jnp.float32),
    )(x, p0, p1, maskf)


def _mlp_body(x_ref, p0_ref, p1_ref, w1_ref, b1_ref, g_ref, be_ref,
              w2_ref, b2_ref, o_ref):
    h = x_ref[...] + p0_ref[...] + p1_ref[...]
    z = jnp.dot(h, w1_ref[...], preferred_element_type=jnp.float32)
    z = (z + b1_ref[...]) * (g_ref[...] * BN_S) + be_ref[...]
    a = jnp.maximum(z, 0.0)
    z2 = jnp.dot(a, w2_ref[...], preferred_element_type=jnp.float32)
    o_ref[...] = jnp.maximum(z2 + b2_ref[...], 0.0)


def _tc_mlp(x, p0, p1, w1, b1, g, be, w2, b2):
    return pl.pallas_call(
        _mlp_body,
        grid=(NBLK,),
        in_specs=[_ROW, _ROW, _ROW,
                  _FULL_W, _FULL_B, _FULL_B, _FULL_B, _FULL_W, _FULL_B],
        out_specs=_ROW,
        out_shape=jax.ShapeDtypeStruct((N, D), jnp.float32),
    )(x, p0, p1, w1, b1[None], g[None], be[None], w2, b2[None])


def _mlp_pool_body(x_ref, p0_ref, p1_ref, w1_ref, b1_ref, g_ref, be_ref,
                   w2_ref, b2_ref, batch_ref, fw1_ref, fb1_ref, fw2_ref,
                   fb2_ref, o_ref, pooled):
    i = pl.program_id(0)
    h = x_ref[...] + p0_ref[...] + p1_ref[...]
    z = jnp.dot(h, w1_ref[...], preferred_element_type=jnp.float32)
    z = (z + b1_ref[...]) * (g_ref[...] * BN_S) + be_ref[...]
    a = jnp.maximum(z, 0.0)
    z2 = jnp.dot(a, w2_ref[...], preferred_element_type=jnp.float32)
    xl = jnp.maximum(z2 + b2_ref[...], 0.0)

    gids = lax.broadcasted_iota(jnp.int32, (1, G), 1)
    oh = (batch_ref[...] == gids).astype(jnp.float32)
    contrib = lax.dot_general(oh, xl, (((0,), (0,)), ((), ())),
                              preferred_element_type=jnp.float32)

    @pl.when(i == 0)
    def _():
        pooled[...] = jnp.zeros_like(pooled)

    pooled[...] += contrib

    @pl.when(i == NBLK - 1)
    def _():
        hh = jnp.dot(pooled[...], fw1_ref[...],
                     preferred_element_type=jnp.float32)
        hh = jnp.maximum(hh + fb1_ref[...], 0.0)
        res = jnp.dot(hh, fw2_ref[...],
                      preferred_element_type=jnp.float32) + fb2_ref[...]
        o_ref[...] = res[:, :C]


def _tc_mlp_pool(x, p0, p1, w1, b1, g, be, w2, b2, batch2d, fw1, fb1,
                 fw2p, fb2p):
    return pl.pallas_call(
        _mlp_pool_body,
        grid=(NBLK,),
        in_specs=[_ROW, _ROW, _ROW,
                  _FULL_W, _FULL_B, _FULL_B, _FULL_B, _FULL_W, _FULL_B,
                  pl.BlockSpec((BLK, 1), lambda i: (i, 0)),
                  _FULL_W, _FULL_B, _FULL_W, _FULL_B],
        out_specs=pl.BlockSpec((G, C), lambda i: (0, 0)),
        out_shape=jax.ShapeDtypeStruct((G, C), jnp.float32),
        scratch_shapes=[pltpu.VMEM((G, D), jnp.float32)],
    )(x, p0, p1, w1, b1[None], g[None], be[None], w2, b2[None], batch2d,
      fw1, fb1[None], fw2p, fb2p[None])


def kernel(x, edge_index, supernode_mask, edge_mask, batch,
           W1_0, b1_0, g_0, be_0, W2_0, b2_0,
           W1_1, b1_1, g_1, be_1, W2_1, b2_1,
           W1_2, b1_2, g_2, be_2, W2_2, b2_2,
           fW1, fb1, fW2, fb2):
    src = edge_index[0]
    dst = edge_index[1]
    maskf = supernode_mask.astype(jnp.float32)[:, None]
    batch2d = batch[:, None]
    fW2p = jnp.pad(fW2, ((0, 0), (0, D - C)))
    fb2p = jnp.pad(fb2, (0, D - C))

    p0, p1 = _sc_seg_w(x, src, dst, edge_mask)
    x1 = _tc_select(x, p0, p1, maskf)

    q0, q1 = _sc_seg(x1, src, dst)
    x2 = _tc_mlp(x1, q0, q1, W1_0, b1_0, g_0, be_0, W2_0, b2_0)

    r0, r1 = _sc_seg(x2, src, dst)
    x3 = _tc_mlp(x2, r0, r1, W1_1, b1_1, g_1, be_1, W2_1, b2_1)

    t0, t1 = _sc_seg(x3, src, dst)
    return _tc_mlp_pool(x3, t0, t1, W1_2, b1_2, g_2, be_2, W2_2, b2_2,
                        batch2d, fW1, fb1, fW2p, fb2p)
